# baseline (speedup 1.0000x reference)
import functools

import jax
import jax.numpy as jnp
from jax import lax
from jax.experimental import pallas as pl
from jax.experimental.pallas import tpu as pltpu
from jax.experimental.pallas import tpu_sc as plsc

ALPHA = 0.2
WIN = 4
DEG = 32
SDEG = 33
D = 128
NW = 32
M_PAD = 10240
R0 = 384
R1 = 256
IDX_ROWS = 10368
CN = 32


def _tc_body(x_ref, w_ref, c_ref, h_ref, s_ref):
    h = jnp.dot(x_ref[...], w_ref[...], preferred_element_type=jnp.float32)
    h_ref[...] = h
    s_ref[...] = jnp.dot(h, c_ref[...], preferred_element_type=jnp.float32)


def _tc_matmul(x, w, c12):
    m = x.shape[0]
    bm = 2000
    return pl.pallas_call(
        _tc_body,
        grid=(m // bm,),
        in_specs=[
            pl.BlockSpec((bm, D), lambda i: (i, 0)),
            pl.BlockSpec((D, D), lambda i: (0, 0)),
            pl.BlockSpec((D, 2), lambda i: (0, 0)),
        ],
        out_specs=[
            pl.BlockSpec((bm, D), lambda i: (i, 0)),
            pl.BlockSpec((bm, 2), lambda i: (i, 0)),
        ],
        out_shape=[
            jax.ShapeDtypeStruct((m, D), jnp.float32),
            jax.ShapeDtypeStruct((m, 2), jnp.float32),
        ],
    )(x, w, c12)


def _sc_attention(h, s_flat, idx_flat):
    mesh = plsc.VectorSubcoreMesh(core_axis_name="c", subcore_axis_name="s")

    @functools.partial(
        pl.kernel,
        mesh=mesh,
        compiler_params=pltpu.CompilerParams(needs_layout_passes=False),
        out_type=jax.ShapeDtypeStruct((M_PAD * D,), jnp.float32),
        scratch_types=[
            pltpu.VMEM((2 * M_PAD,), jnp.float32),
            pltpu.VMEM((R0 * SDEG,), jnp.int32),
            pltpu.VMEM((16 * DEG,), jnp.int32),
            pltpu.VMEM((16 * DEG,), jnp.float32),
            pltpu.VMEM((R0 // CN, CN * WIN), jnp.int32),
            pltpu.VMEM((R0 * WIN,), jnp.float32),
            pltpu.VMEM((CN * WIN, D), jnp.float32),
            pltpu.VMEM((CN * WIN, D), jnp.float32),
            pltpu.VMEM((CN * D,), jnp.float32),
            pltpu.VMEM((CN * D,), jnp.float32),
            pltpu.SemaphoreType.DMA,
            pltpu.SemaphoreType.DMA,
            pltpu.SemaphoreType.DMA,
            pltpu.SemaphoreType.DMA,
        ],
    )
    def k(h_hbm, s_hbm, idx_hbm, out_hbm,
          s_v, idx_v, nidx_v, eb_v, gidx_v, wts_v, rows0_v, rows1_v,
          out0_v, out1_v, sem0, sem1, osem0, osem1):
        cid = lax.axis_index("c")
        sid = lax.axis_index("s")
        base = jnp.where(cid == 0, sid * R0, 16 * R0 + sid * R1)
        nbody = jnp.where(cid == 0, R0 // (4 * 16), R1 // (4 * 16))
        last_chunk = jnp.where(cid == 0, R0 // CN - 1, R1 // CN - 1)
        pltpu.sync_copy(s_hbm, s_v)
        pltpu.sync_copy(idx_hbm.at[pl.ds(base * SDEG, R0 * SDEG)], idx_v)

        lane = lax.iota(jnp.int32, 16)

        def batch_body(b, carry):
            node16 = b * 16 + lane
            jbase = b * (16 * SDEG) + lane * SDEG
            s1b = plsc.load_gather(s_v, [(base + node16) * 2])
            for j in range(DEG):
                nid = plsc.load_gather(idx_v, [jbase + j])
                nidx_v[pl.ds(j * 16, 16)] = nid
            e_max = jnp.full((16,), -jnp.inf, jnp.float32)
            for j in range(DEG):
                nid = nidx_v[pl.ds(j * 16, 16)]
                s2g = plsc.load_gather(s_v, [nid * 2 + 1])
                e = s1b + s2g
                e = jnp.where(e >= 0.0, e, ALPHA * e)
                e_max = jnp.maximum(e_max, e)
                eb_v[pl.ds(j * 16, 16)] = e
            ssum = jnp.zeros((16,), jnp.float32)
            cs = jnp.zeros((16,), jnp.float32)
            cs_list = []
            best = None
            bestj = None
            for j in range(DEG):
                p = jnp.exp(eb_v[pl.ds(j * 16, 16)] - e_max)
                eb_v[pl.ds(j * 16, 16)] = p
                ssum = ssum + p
                cs = cs + p
                cs_list.append(cs)
                if j == WIN - 1:
                    best = cs
                    bestj = jnp.full((16,), j, jnp.int32)
                elif j > WIN - 1:
                    w = cs - cs_list[j - WIN]
                    upd = w > best
                    best = jnp.where(upd, w, best)
                    bestj = jnp.where(upd, j, bestj)
            recip = (float(DEG) / float(WIN)) / ssum
            start = bestj - (WIN - 1)
            for i in range(WIN):
                pos = start + i
                pg = plsc.load_gather(eb_v, [pos * 16 + lane])
                wt = pg * recip
                q = node16 * WIN + i
                plsc.store_scatter(wts_v, [q], wt)
                nbr = plsc.load_gather(idx_v, [node16 * SDEG + pos])
                plsc.store_scatter(
                    gidx_v, [jnp.right_shift(q, 7), jnp.bitwise_and(q, 127)], nbr)
            return carry

        def gather_dma(c, buf, sem):
            return pltpu.make_async_copy(h_hbm.at[gidx_v.at[c]], buf, sem)

        def out_dma(c, obuf, sem):
            return pltpu.make_async_copy(
                obuf, out_hbm.at[pl.ds((base + c * CN) * D, CN * D)], sem)

        def combine(c, buf, obuf, osem):
            @pl.when(c >= 2)
            def _():
                out_dma(c - 2, obuf, osem).wait()

            def nl_body(nl, carry):
                wrow = wts_v[pl.ds(c * (CN * WIN) + nl * WIN, 16)]
                wv = [wrow.at[jnp.full((16,), i, jnp.int32)].get(
                          mode="promise_in_bounds")
                      for i in range(WIN)]
                for ch in range(D // 16):
                    acc = wv[0] * buf[nl * WIN + 0, pl.ds(ch * 16, 16)]
                    for i in range(1, WIN):
                        acc = acc + wv[i] * buf[nl * WIN + i, pl.ds(ch * 16, 16)]
                    obuf[pl.ds(nl * D + ch * 16, 16)] = jnp.where(
                        acc > 0.0, acc, jnp.exp(acc) - 1.0)
                return carry

            lax.fori_loop(0, CN, nl_body, 0)
            out_dma(c, obuf, osem).start()

        def pipe_body(i, carry):
            b0 = 4 * i
            lax.fori_loop(b0, b0 + 2, batch_body, 0)
            gather_dma(2 * i, rows0_v, sem0).start()

            @pl.when(i > 0)
            def _():
                gather_dma(2 * i - 1, rows1_v, sem1).wait()
                combine(2 * i - 1, rows1_v, out1_v, osem1)

            lax.fori_loop(b0 + 2, b0 + 4, batch_body, 0)
            gather_dma(2 * i + 1, rows1_v, sem1).start()
            gather_dma(2 * i, rows0_v, sem0).wait()
            combine(2 * i, rows0_v, out0_v, osem0)
            return carry

        lax.fori_loop(0, nbody, pipe_body, 0)
        gather_dma(last_chunk, rows1_v, sem1).wait()
        combine(last_chunk, rows1_v, out1_v, osem1)
        out_dma(last_chunk - 1, out0_v, osem0).wait()
        out_dma(last_chunk, out1_v, osem1).wait()

    return k(h, s_flat, idx_flat)


def kernel(input, adj, M, W, c1, c2):
    m0, d_in = input.shape
    c12 = jnp.concatenate([c1, c2], axis=1)
    h, s = _tc_matmul(input, W, c12)
    s_flat = jnp.pad(s, ((0, M_PAD - m0), (0, 0))).reshape(-1)
    idx = adj.reshape(m0, DEG).astype(jnp.int32)
    idx_flat = jnp.pad(idx, ((0, IDX_ROWS - m0), (0, SDEG - DEG))).reshape(-1)
    out_flat = _sc_attention(h, s_flat, idx_flat)
    return out_flat.reshape(M_PAD, D)[:m0]


# speedup vs baseline: 1.0628x; 1.0628x over previous
import functools

import jax
import jax.numpy as jnp
from jax import lax
from jax.experimental import pallas as pl
from jax.experimental.pallas import tpu as pltpu
from jax.experimental.pallas import tpu_sc as plsc

ALPHA = 0.2
WIN = 4
DEG = 32
SDEG = 33
D = 128
NW = 32
M_PAD = 10240
R0 = 384
R1 = 256
IDX_ROWS = 10368
CN = 32


def _tc_body(x_ref, w_ref, c_ref, h_ref, s_ref):
    h = jnp.dot(x_ref[...], w_ref[...], preferred_element_type=jnp.float32)
    h_ref[...] = h
    s_ref[...] = jnp.dot(h, c_ref[...], preferred_element_type=jnp.float32)


def _tc_matmul(x, w, c12):
    m = x.shape[0]
    bm = 2000
    return pl.pallas_call(
        _tc_body,
        grid=(m // bm,),
        in_specs=[
            pl.BlockSpec((bm, D), lambda i: (i, 0)),
            pl.BlockSpec((D, D), lambda i: (0, 0)),
            pl.BlockSpec((D, 2), lambda i: (0, 0)),
        ],
        out_specs=[
            pl.BlockSpec((bm, D), lambda i: (i, 0)),
            pl.BlockSpec((bm, 2), lambda i: (i, 0)),
        ],
        out_shape=[
            jax.ShapeDtypeStruct((m, D), jnp.float32),
            jax.ShapeDtypeStruct((m, 2), jnp.float32),
        ],
    )(x, w, c12)


def _sc_attention(h, s_flat, idx_flat):
    mesh = plsc.VectorSubcoreMesh(core_axis_name="c", subcore_axis_name="s")

    @functools.partial(
        pl.kernel,
        mesh=mesh,
        compiler_params=pltpu.CompilerParams(needs_layout_passes=False),
        out_type=jax.ShapeDtypeStruct((M_PAD * D,), jnp.float32),
        scratch_types=[
            pltpu.VMEM((2 * M_PAD,), jnp.float32),
            pltpu.VMEM((R0 * SDEG,), jnp.int32),
            pltpu.VMEM((16 * DEG,), jnp.int32),
            pltpu.VMEM((16 * DEG,), jnp.float32),
            pltpu.VMEM((R0 // CN, CN * WIN), jnp.int32),
            pltpu.VMEM((R0 * WIN,), jnp.float32),
            pltpu.VMEM((CN * WIN, D), jnp.float32),
            pltpu.VMEM((CN * WIN, D), jnp.float32),
            pltpu.VMEM((CN * D,), jnp.float32),
            pltpu.VMEM((CN * D,), jnp.float32),
            pltpu.SemaphoreType.DMA,
            pltpu.SemaphoreType.DMA,
            pltpu.SemaphoreType.DMA,
            pltpu.SemaphoreType.DMA,
        ],
    )
    def k(h_hbm, s_hbm, idx_hbm, out_hbm,
          s_v, idx_v, nidx_v, eb_v, gidx_v, wts_v, rows0_v, rows1_v,
          out0_v, out1_v, sem0, sem1, osem0, osem1):
        cid = lax.axis_index("c")
        sid = lax.axis_index("s")
        base = jnp.where(cid == 0, sid * R0, 16 * R0 + sid * R1)
        nbody = jnp.where(cid == 0, R0 // (4 * 16), R1 // (4 * 16))
        last_chunk = jnp.where(cid == 0, R0 // CN - 1, R1 // CN - 1)
        pltpu.sync_copy(s_hbm, s_v)
        pltpu.sync_copy(idx_hbm.at[pl.ds(base * SDEG, R0 * SDEG)], idx_v)

        lane = lax.iota(jnp.int32, 16)

        def batch_body(b, carry):
            node16 = b * 16 + lane
            jbase = b * (16 * SDEG) + lane * SDEG
            s1b = plsc.load_gather(s_v, [(base + node16) * 2])
            for j in range(DEG):
                nid = plsc.load_gather(idx_v, [jbase + j])
                nidx_v[pl.ds(j * 16, 16)] = nid
            e_max = jnp.full((16,), -jnp.inf, jnp.float32)
            for j in range(DEG):
                nid = nidx_v[pl.ds(j * 16, 16)]
                s2g = plsc.load_gather(s_v, [nid * 2 + 1])
                e = s1b + s2g
                e = jnp.where(e >= 0.0, e, ALPHA * e)
                e_max = jnp.maximum(e_max, e)
                eb_v[pl.ds(j * 16, 16)] = e
            ssum = jnp.zeros((16,), jnp.float32)
            cs = jnp.zeros((16,), jnp.float32)
            cs_list = []
            best = None
            bestj = None
            for j in range(DEG):
                p = jnp.exp(eb_v[pl.ds(j * 16, 16)] - e_max)
                eb_v[pl.ds(j * 16, 16)] = p
                ssum = ssum + p
                cs = cs + p
                cs_list.append(cs)
                if j == WIN - 1:
                    best = cs
                    bestj = jnp.full((16,), j, jnp.int32)
                elif j > WIN - 1:
                    w = cs - cs_list[j - WIN]
                    upd = w > best
                    best = jnp.where(upd, w, best)
                    bestj = jnp.where(upd, j, bestj)
            recip = (float(DEG) / float(WIN)) / ssum
            start = bestj - (WIN - 1)
            for i in range(WIN):
                pos = start + i
                pg = plsc.load_gather(eb_v, [pos * 16 + lane])
                wt = pg * recip
                q = node16 * WIN + i
                plsc.store_scatter(wts_v, [q], wt)
                nbr = plsc.load_gather(idx_v, [node16 * SDEG + pos])
                plsc.store_scatter(
                    gidx_v, [jnp.right_shift(q, 7), jnp.bitwise_and(q, 127)], nbr)
            return carry

        def gather_dma(c, buf, sem):
            return pltpu.make_async_copy(h_hbm.at[gidx_v.at[c]], buf, sem)

        def out_dma(c, obuf, sem):
            return pltpu.make_async_copy(
                obuf, out_hbm.at[pl.ds((base + c * CN) * D, CN * D)], sem)

        def combine(c, buf, obuf, osem):
            @pl.when(c >= 2)
            def _():
                out_dma(c - 2, obuf, osem).wait()

            for nl in range(CN):
                if nl % 4 == 0:
                    wrow = wts_v[pl.ds(c * (CN * WIN) + nl * WIN, 16)]
                k0 = (nl % 4) * WIN
                wv = [wrow.at[jnp.full((16,), k0 + i, jnp.int32)].get(
                          mode="promise_in_bounds")
                      for i in range(WIN)]
                for ch in range(D // 16):
                    acc = wv[0] * buf[nl * WIN + 0, pl.ds(ch * 16, 16)]
                    for i in range(1, WIN):
                        acc = acc + wv[i] * buf[nl * WIN + i, pl.ds(ch * 16, 16)]
                    obuf[pl.ds(nl * D + ch * 16, 16)] = jnp.where(
                        acc > 0.0, acc, jnp.exp(acc) - 1.0)
            out_dma(c, obuf, osem).start()

        def pipe_body(i, carry):
            b0 = 4 * i
            batch_body(b0, 0)
            batch_body(b0 + 1, 0)
            gather_dma(2 * i, rows0_v, sem0).start()

            @pl.when(i > 0)
            def _():
                gather_dma(2 * i - 1, rows1_v, sem1).wait()
                combine(2 * i - 1, rows1_v, out1_v, osem1)

            batch_body(b0 + 2, 0)
            batch_body(b0 + 3, 0)
            gather_dma(2 * i + 1, rows1_v, sem1).start()
            gather_dma(2 * i, rows0_v, sem0).wait()
            combine(2 * i, rows0_v, out0_v, osem0)
            return carry

        lax.fori_loop(0, nbody, pipe_body, 0)
        gather_dma(last_chunk, rows1_v, sem1).wait()
        combine(last_chunk, rows1_v, out1_v, osem1)
        out_dma(last_chunk - 1, out0_v, osem0).wait()
        out_dma(last_chunk, out1_v, osem1).wait()

    return k(h, s_flat, idx_flat)


def kernel(input, adj, M, W, c1, c2):
    m0, d_in = input.shape
    c12 = jnp.concatenate([c1, c2], axis=1)
    h, s = _tc_matmul(input, W, c12)
    s_flat = jnp.pad(s, ((0, M_PAD - m0), (0, 0))).reshape(-1)
    idx = adj.reshape(m0, DEG).astype(jnp.int32)
    idx_flat = jnp.pad(idx, ((0, IDX_ROWS - m0), (0, SDEG - DEG))).reshape(-1)
    out_flat = _sc_attention(h, s_flat, idx_flat)
    return out_flat.reshape(M_PAD, D)[:m0]
